# baseline (speedup 1.0000x reference)
import functools

import jax
import jax.numpy as jnp
from jax import lax
from jax.experimental import pallas as pl
from jax.experimental.pallas import tpu as pltpu
from jax.experimental.pallas import tpu_sc as plsc

_NC = 2
_NS = 16
_CH = 80
_NG = 128


def _sc_degree(n, e):
    ept = e // (_NC * _NS)
    mesh = plsc.VectorSubcoreMesh(core_axis_name="c", subcore_axis_name="s")

    @functools.partial(
        pl.kernel,
        out_type=jax.ShapeDtypeStruct((_NC * _NS, n), jnp.float32),
        mesh=mesh,
        scratch_types=[
            pltpu.VMEM((ept,), jnp.int32),
            pltpu.VMEM((n,), jnp.float32),
        ],
        compiler_params=pltpu.CompilerParams(needs_layout_passes=False),
    )
    def deg_kernel(dst_hbm, out_hbm, dst_v, deg_v):
        c = lax.axis_index("c")
        s = lax.axis_index("s")
        wid = c * _NS + s
        pltpu.sync_copy(dst_hbm.at[pl.ds(wid * ept, ept)], dst_v)

        def zero_body(i, carry):
            deg_v[pl.ds(i * 16, 16)] = jnp.zeros((16,), jnp.float32)
            return carry

        lax.fori_loop(0, n // 16, zero_body, 0)

        ones = jnp.ones((16,), jnp.float32)

        def body(i, carry):
            idx = dst_v[pl.ds(i * 16, 16)]
            plsc.addupdate_scatter(deg_v, [idx], ones)
            return carry

        lax.fori_loop(0, ept // 16, body, 0)
        pltpu.sync_copy(deg_v, out_hbm.at[wid])

    return deg_kernel


def _sc_prop(n, d, e, branch_split):
    ept = e // _NS if branch_split else e // (_NC * _NS)
    sb = 2000
    nsb = ept // sb
    ncs = sb // _CH
    npt = (n // (8 * _NS)) * 8
    rem = n - npt * _NS
    mesh = plsc.VectorSubcoreMesh(core_axis_name="c", subcore_axis_name="s")

    @functools.partial(
        pl.kernel,
        out_type=jax.ShapeDtypeStruct((_NC, n, d), jnp.float32),
        mesh=mesh,
        scratch_types=[
            pltpu.VMEM((sb,), jnp.int32),
            pltpu.VMEM((sb,), jnp.int32),
            pltpu.VMEM((_CH, d), jnp.float32),
            pltpu.VMEM((_CH, d), jnp.float32),
            pltpu.VMEM((_CH, d), jnp.float32),
            pltpu.VMEM((_CH, d), jnp.float32),
            pltpu.VMEM((_CH,), jnp.int32),
            pltpu.VMEM((_CH,), jnp.int32),
            pltpu.VMEM_SHARED((n, d), jnp.float32),
            [pltpu.SemaphoreType.DMA] * 4,
            [pltpu.SemaphoreType.DMA] * 4,
            pltpu.SemaphoreType.DMA,
        ],
    )
    def prop_kernel(src_hbm, dst_hbm, h_hbm, zero_hbm, out_hbm,
                    src_v, dst_v, buf0, buf1, buf2, buf3, di0, di1, acc,
                    gsem, ssem, semz):
        c = lax.axis_index("c")
        s = lax.axis_index("s")

        pltpu.async_copy(zero_hbm.at[pl.ds(s * npt, npt)],
                         acc.at[pl.ds(s * npt, npt)], semz)
        if rem:
            @pl.when(s == 0)
            def _():
                pltpu.async_copy(zero_hbm.at[pl.ds(npt * _NS, rem)],
                                 acc.at[pl.ds(npt * _NS, rem)], semz)

        if branch_split:
            base = s * ept
        else:
            base = (c * _NS + s) * ept
        off = c * n

        def zero_wait():
            pltpu.make_async_copy(zero_hbm.at[pl.ds(s * npt, npt)],
                                  acc.at[pl.ds(s * npt, npt)], semz).wait()
            if rem:
                @pl.when(s == 0)
                def _():
                    pltpu.make_async_copy(
                        zero_hbm.at[pl.ds(npt * _NS, rem)],
                        acc.at[pl.ds(npt * _NS, rem)], semz).wait()

        bufs = (buf0, buf1, buf2, buf3)
        dis = (di0, di1)
        nslot = len(bufs)

        def issue(k, p):
            pltpu.async_copy(h_hbm.at[src_v.at[pl.ds(k * _CH, _CH)]],
                             bufs[p], gsem[p])

        def gwait(p):
            pltpu.make_async_copy(h_hbm.at[pl.ds(0, _CH)], bufs[p],
                                  gsem[p]).wait()

        def swait(p):
            pltpu.make_async_copy(bufs[p], acc.at[pl.ds(0, _CH)],
                                  ssem[p]).wait()

        def step(k, u, kk):
            gwait(u)
            di = dis[u % 2]
            for j in range(_CH // 16):
                di[pl.ds(j * 16, 16)] = dst_v[pl.ds(k * _CH + j * 16, 16)]
            pltpu.async_copy(bufs[u], acc.at[di], ssem[u], add=True)
            q = (u + nslot - 1) % nslot
            if kk is None:
                swait(q)
            else:
                if u == 0:
                    @pl.when(kk > 0)
                    def _():
                        swait(q)
                else:
                    swait(q)

                @pl.when(k + nslot - 1 < ncs)
                def _():
                    issue(k + nslot - 1, q)

        def sb_body(b, carry):
            pltpu.sync_copy(src_hbm.at[pl.ds(base + b * sb, sb)], src_v)
            pltpu.sync_copy(dst_hbm.at[pl.ds(base + b * sb, sb)], dst_v)
            if branch_split:
                def offs_body(i, c2):
                    src_v[pl.ds(i * 16, 16)] = src_v[pl.ds(i * 16, 16)] + off
                    return c2

                lax.fori_loop(0, sb // 16, offs_body, 0)

            for p in range(nslot - 1):
                issue(p, p)

            @pl.when(b == 0)
            def _():
                zero_wait()
                plsc.subcore_barrier()

            def body(kk, c2):
                for u in range(nslot):
                    step(nslot * kk + u, u, kk)
                return c2

            lax.fori_loop(0, ncs // nslot, body, 0)
            ntail = ncs - (ncs // nslot) * nslot
            for t in range(ntail):
                step((ncs // nslot) * nslot + t, t, None)
            swait((ncs - 1) % nslot)
            return carry

        lax.fori_loop(0, nsb, sb_body, 0)

        plsc.subcore_barrier()
        pltpu.sync_copy(acc.at[pl.ds(s * npt, npt)],
                        out_hbm.at[c, pl.ds(s * npt, npt)])
        if rem:
            @pl.when(s == 0)
            def _():
                pltpu.sync_copy(acc.at[pl.ds(npt * _NS, rem)],
                                out_hbm.at[c, pl.ds(npt * _NS, rem)])

    return prop_kernel


_ARB = pltpu.CompilerParams(dimension_semantics=("arbitrary",))
_ARB2 = pltpu.CompilerParams(dimension_semantics=("arbitrary", "arbitrary"))


def _inv_col(degp):
    ones = jnp.ones((_NC * _NS, 1), jnp.float32)
    deg = lax.dot_general(degp, ones, (((0,), (0,)), ((), ())),
                          preferred_element_type=jnp.float32)
    return lax.rsqrt(deg + 1.0)


def _tc_prep(n, d, nb, r):
    def kern(degp_ref, x_ref, hs_ref):
        inv = _inv_col(degp_ref[...])
        hs_ref[...] = x_ref[...] * inv

    return pl.pallas_call(
        kern,
        grid=(nb,),
        in_specs=[
            pl.BlockSpec((_NC * _NS, r), lambda i: (0, i)),
            pl.BlockSpec((r, d), lambda i: (i, 0)),
        ],
        out_specs=pl.BlockSpec((r, d), lambda i: (i, 0)),
        out_shape=jax.ShapeDtypeStruct((n, d), jnp.float32),
        compiler_params=_ARB,
    )


def _tc_layer1(n, d, nb, r):
    def kern(aggp_ref, x_ref, degp_ref, w_ref, b_ref, z_ref, hz_ref):
        p = aggp_ref[...]
        inv = _inv_col(degp_ref[...])
        x = x_ref[...]
        agg = (p[0] + p[1]) * inv + x * inv * inv
        z = jnp.maximum(
            jnp.dot(agg, w_ref[0], preferred_element_type=jnp.float32)
            + b_ref[0], 0.0)
        z_ref[...] = z[None]
        hz_ref[...] = (z * inv)[None]

    return pl.pallas_call(
        kern,
        grid=(2, nb),
        in_specs=[
            pl.BlockSpec((2, r, d), lambda br, i: (0, i, 0)),
            pl.BlockSpec((r, d), lambda br, i: (i, 0)),
            pl.BlockSpec((_NC * _NS, r), lambda br, i: (0, i)),
            pl.BlockSpec((1, d, d), lambda br, i: (br, 0, 0)),
            pl.BlockSpec((1, 1, d), lambda br, i: (br, 0, 0)),
        ],
        out_specs=[
            pl.BlockSpec((1, r, d), lambda br, i: (br, i, 0)),
            pl.BlockSpec((1, r, d), lambda br, i: (br, i, 0)),
        ],
        out_shape=[
            jax.ShapeDtypeStruct((2, n, d), jnp.float32),
            jax.ShapeDtypeStruct((2, n, d), jnp.float32),
        ],
        compiler_params=_ARB2,
    )


def _tc_final(n, d, nb, r):
    def kern(agg2_ref, z1_ref, degp_ref, batch_ref, w2_ref, b2_ref,
             m1w1_ref, m1b1_ref, m1w2_ref, m1b2_ref,
             m2w1_ref, m2b1_ref, m2w2_ref, m2b2_ref,
             h1_ref, h2_ref, g1_ref, g2_ref, gacc0, gacc1, cnt):
        i = pl.program_id(0)
        br = pl.program_id(1)
        nblocks = pl.num_programs(0)

        @pl.when((i == 0) & (br == 0))
        def _():
            gacc0[...] = jnp.zeros_like(gacc0)
            gacc1[...] = jnp.zeros_like(gacc1)
            cnt[...] = jnp.zeros_like(cnt)

        inv = _inv_col(degp_ref[...])
        z1 = z1_ref[0]
        agg = agg2_ref[0] * inv + z1 * inv * inv
        z2 = jnp.maximum(
            jnp.dot(agg, w2_ref[0], preferred_element_type=jnp.float32)
            + b2_ref[0], 0.0)

        t = jnp.maximum(
            jnp.dot(z2, m1w1_ref[...], preferred_element_type=jnp.float32)
            + m1b1_ref[...], 0.0)
        h = (jnp.dot(t, m1w2_ref[...], preferred_element_type=jnp.float32)
             + m1b2_ref[...])

        @pl.when(br == 0)
        def _():
            h1_ref[...] = h

        @pl.when(br == 1)
        def _():
            h2_ref[...] = h

        b = batch_ref[...]
        gids = lax.broadcasted_iota(jnp.int32, (_NG, r), 0)
        rows = lax.broadcasted_iota(jnp.int32, (1, r), 1) + i * r
        onehot = jnp.where((gids == b) & (rows < n), 1.0, 0.0)
        gpart = jnp.dot(onehot, z2, preferred_element_type=jnp.float32)

        @pl.when(br == 0)
        def _():
            gacc0[...] += gpart
            cnt[...] += jnp.broadcast_to(
                jnp.sum(onehot, axis=1, keepdims=True), (_NG, d))

        @pl.when(br == 1)
        def _():
            gacc1[...] += gpart

        @pl.when((i == nblocks - 1) & (br == 1))
        def _():
            c = jnp.maximum(cnt[...], 1.0)
            for g_ref, acc in ((g1_ref, gacc0), (g2_ref, gacc1)):
                gg = acc[...] / c
                gg = jnp.maximum(
                    jnp.dot(gg, m2w1_ref[...],
                            preferred_element_type=jnp.float32)
                    + m2b1_ref[...], 0.0)
                g_ref[...] = (jnp.dot(gg, m2w2_ref[...],
                                      preferred_element_type=jnp.float32)
                              + m2b2_ref[...])

    return pl.pallas_call(
        kern,
        grid=(nb, 2),
        in_specs=[
            pl.BlockSpec((1, r, d), lambda i, br: (br, i, 0)),
            pl.BlockSpec((1, r, d), lambda i, br: (br, i, 0)),
            pl.BlockSpec((_NC * _NS, r), lambda i, br: (0, i)),
            pl.BlockSpec((1, r), lambda i, br: (0, i)),
            pl.BlockSpec((1, d, d), lambda i, br: (br, 0, 0)),
            pl.BlockSpec((1, 1, d), lambda i, br: (br, 0, 0)),
            pl.BlockSpec((d, d), lambda i, br: (0, 0)),
            pl.BlockSpec((1, d), lambda i, br: (0, 0)),
            pl.BlockSpec((d, d), lambda i, br: (0, 0)),
            pl.BlockSpec((1, d), lambda i, br: (0, 0)),
            pl.BlockSpec((d, d), lambda i, br: (0, 0)),
            pl.BlockSpec((1, d), lambda i, br: (0, 0)),
            pl.BlockSpec((d, d), lambda i, br: (0, 0)),
            pl.BlockSpec((1, d), lambda i, br: (0, 0)),
        ],
        out_specs=[
            pl.BlockSpec((r, d), lambda i, br: (i, 0)),
            pl.BlockSpec((r, d), lambda i, br: (i, 0)),
            pl.BlockSpec((_NG, d), lambda i, br: (0, 0)),
            pl.BlockSpec((_NG, d), lambda i, br: (0, 0)),
        ],
        out_shape=[
            jax.ShapeDtypeStruct((n, d), jnp.float32),
            jax.ShapeDtypeStruct((n, d), jnp.float32),
            jax.ShapeDtypeStruct((_NG, d), jnp.float32),
            jax.ShapeDtypeStruct((_NG, d), jnp.float32),
        ],
        scratch_shapes=[
            pltpu.VMEM((_NG, d), jnp.float32),
            pltpu.VMEM((_NG, d), jnp.float32),
            pltpu.VMEM((_NG, d), jnp.float32),
        ],
        compiler_params=_ARB2,
    )


def kernel(x, edge_index, batch,
           gcn1_W1, gcn1_b1, gcn1_W2, gcn1_b2,
           gcn2_W1, gcn2_b1, gcn2_W2, gcn2_b2,
           mlp1_W1, mlp1_b1, mlp1_W2, mlp1_b2,
           mlp2_W1, mlp2_b1, mlp2_W2, mlp2_b2):
    n, d = x.shape
    e = edge_index.shape[1]
    r = 1024
    nb = pl.cdiv(n, r)

    src = edge_index[0].astype(jnp.int32)
    dst = edge_index[1].astype(jnp.int32)

    degp = _sc_degree(n, e)(dst)
    hs = _tc_prep(n, d, nb, r)(degp, x)

    zero = jnp.zeros((n, d), jnp.float32)
    aggp = _sc_prop(n, d, e, False)(src, dst, hs, zero)

    w1s = jnp.stack([gcn1_W1, gcn2_W1])
    b1s = jnp.stack([gcn1_b1, gcn2_b1])[:, None, :]
    z, hz = _tc_layer1(n, d, nb, r)(aggp, x, degp, w1s, b1s)

    agg2 = _sc_prop(n, d, e, True)(src, dst, hz.reshape(2 * n, d), zero)

    w2s = jnp.stack([gcn1_W2, gcn2_W2])
    b2s = jnp.stack([gcn1_b2, gcn2_b2])[:, None, :]
    h1, h2, g1, g2 = _tc_final(n, d, nb, r)(
        agg2, z, degp, batch.astype(jnp.int32)[None], w2s, b2s,
        mlp1_W1, mlp1_b1[None], mlp1_W2, mlp1_b2[None],
        mlp2_W1, mlp2_b1[None], mlp2_W2, mlp2_b2[None])

    return h1, h2, g1, g2


# speedup vs baseline: 1.0515x; 1.0515x over previous
import functools

import jax
import jax.numpy as jnp
from jax import lax
from jax.experimental import pallas as pl
from jax.experimental.pallas import tpu as pltpu
from jax.experimental.pallas import tpu_sc as plsc

_NC = 2
_NS = 16
_CH = 80
_NG = 128


def _sc_degree(n, e):
    ept = e // (_NC * _NS)
    mesh = plsc.VectorSubcoreMesh(core_axis_name="c", subcore_axis_name="s")

    @functools.partial(
        pl.kernel,
        out_type=jax.ShapeDtypeStruct((_NC * _NS, n), jnp.float32),
        mesh=mesh,
        scratch_types=[
            pltpu.VMEM((ept,), jnp.int32),
            pltpu.VMEM((n,), jnp.float32),
        ],
        compiler_params=pltpu.CompilerParams(needs_layout_passes=False),
    )
    def deg_kernel(dst_hbm, out_hbm, dst_v, deg_v):
        c = lax.axis_index("c")
        s = lax.axis_index("s")
        wid = c * _NS + s
        pltpu.sync_copy(dst_hbm.at[pl.ds(wid * ept, ept)], dst_v)

        def zero_body(i, carry):
            for j in range(4):
                deg_v[pl.ds(i * 64 + j * 16, 16)] = jnp.zeros(
                    (16,), jnp.float32)
            return carry

        lax.fori_loop(0, n // 64, zero_body, 0)
        for j in range(n // 16 - (n // 64) * 4):
            deg_v[pl.ds((n // 64) * 64 + j * 16, 16)] = jnp.zeros(
                (16,), jnp.float32)

        ones = jnp.ones((16,), jnp.float32)

        def body(i, carry):
            for j in range(4):
                idx = dst_v[pl.ds(i * 64 + j * 16, 16)]
                plsc.addupdate_scatter(deg_v, [idx], ones)
            return carry

        lax.fori_loop(0, ept // 64, body, 0)
        for j in range(ept // 16 - (ept // 64) * 4):
            idx = dst_v[pl.ds((ept // 64) * 64 + j * 16, 16)]
            plsc.addupdate_scatter(deg_v, [idx], ones)
        pltpu.sync_copy(deg_v, out_hbm.at[wid])

    return deg_kernel


def _sc_prop(n, d, e, branch_split):
    ept = e // _NS if branch_split else e // (_NC * _NS)
    sb = 2000
    nsb = ept // sb
    ncs = sb // _CH
    npt = (n // (8 * _NS)) * 8
    rem = n - npt * _NS
    mesh = plsc.VectorSubcoreMesh(core_axis_name="c", subcore_axis_name="s")

    @functools.partial(
        pl.kernel,
        out_type=jax.ShapeDtypeStruct((_NC, n, d), jnp.float32),
        mesh=mesh,
        scratch_types=[
            pltpu.VMEM((sb,), jnp.int32),
            pltpu.VMEM((sb // _CH, _CH), jnp.int32),
            pltpu.VMEM((_CH, d), jnp.float32),
            pltpu.VMEM((_CH, d), jnp.float32),
            pltpu.VMEM((_CH, d), jnp.float32),
            pltpu.VMEM_SHARED((n, d), jnp.float32),
            pltpu.SemaphoreType.DMA,
            pltpu.SemaphoreType.DMA,
            pltpu.SemaphoreType.DMA,
            pltpu.SemaphoreType.DMA,
        ],
    )
    def prop_kernel(src_hbm, dst2_hbm, h_hbm, zero_hbm, out_hbm,
                    src_v, dst_v, buf0, buf1, buf2, acc,
                    sem0, sem1, sem2, semz):
        c = lax.axis_index("c")
        s = lax.axis_index("s")

        pltpu.async_copy(zero_hbm.at[pl.ds(s * npt, npt)],
                         acc.at[pl.ds(s * npt, npt)], semz)
        if rem:
            @pl.when(s == 0)
            def _():
                pltpu.async_copy(zero_hbm.at[pl.ds(npt * _NS, rem)],
                                 acc.at[pl.ds(npt * _NS, rem)], semz)

        if branch_split:
            base = s * ept
            sb_base = s * nsb
        else:
            base = (c * _NS + s) * ept
            sb_base = (c * _NS + s) * nsb
        off = c * n

        def zero_wait():
            pltpu.make_async_copy(zero_hbm.at[pl.ds(s * npt, npt)],
                                  acc.at[pl.ds(s * npt, npt)], semz).wait()
            if rem:
                @pl.when(s == 0)
                def _():
                    pltpu.make_async_copy(
                        zero_hbm.at[pl.ds(npt * _NS, rem)],
                        acc.at[pl.ds(npt * _NS, rem)], semz).wait()

        bufs = (buf0, buf1, buf2)
        sems = (sem0, sem1, sem2)
        nslot = len(bufs)

        def issue(k, p):
            pltpu.async_copy(h_hbm.at[src_v.at[pl.ds(k * _CH, _CH)]],
                             bufs[p], sems[p])

        def wait(p):
            pltpu.make_async_copy(h_hbm.at[pl.ds(0, _CH)], bufs[p],
                                  sems[p]).wait()

        def drain(k, p):
            wait(p)
            pltpu.sync_copy(bufs[p], acc.at[dst_v.at[k]], add=True)

        def sb_body(b, carry):
            pltpu.sync_copy(src_hbm.at[pl.ds(base + b * sb, sb)], src_v)
            pltpu.sync_copy(dst2_hbm.at[sb_base + b], dst_v)
            if branch_split:
                def offs_body(i, c2):
                    src_v[pl.ds(i * 16, 16)] = src_v[pl.ds(i * 16, 16)] + off
                    return c2

                lax.fori_loop(0, sb // 16, offs_body, 0)

            for p in range(nslot):
                issue(p, p)

            @pl.when(b == 0)
            def _():
                zero_wait()
                plsc.subcore_barrier()

            def body(k, c2):
                kk = nslot * k
                for p in range(nslot):
                    drain(kk + p, p)

                    @pl.when(kk + p + nslot < ncs)
                    def _():
                        issue(kk + p + nslot, p)

                return c2

            lax.fori_loop(0, ncs // nslot, body, 0)
            for t in range(ncs - (ncs // nslot) * nslot):
                drain((ncs // nslot) * nslot + t, t)
            return carry

        lax.fori_loop(0, nsb, sb_body, 0)

        plsc.subcore_barrier()
        pltpu.sync_copy(acc.at[pl.ds(s * npt, npt)],
                        out_hbm.at[c, pl.ds(s * npt, npt)])
        if rem:
            @pl.when(s == 0)
            def _():
                pltpu.sync_copy(acc.at[pl.ds(npt * _NS, rem)],
                                out_hbm.at[c, pl.ds(npt * _NS, rem)])

    return prop_kernel


_ARB = pltpu.CompilerParams(dimension_semantics=("arbitrary",))
_ARB2 = pltpu.CompilerParams(dimension_semantics=("arbitrary", "arbitrary"))


def _inv_col(degp):
    ones = jnp.ones((_NC * _NS, 1), jnp.float32)
    deg = lax.dot_general(degp, ones, (((0,), (0,)), ((), ())),
                          preferred_element_type=jnp.float32)
    return lax.rsqrt(deg + 1.0)


def _tc_prep(n, d, nb, r):
    def kern(degp_ref, x_ref, hs_ref):
        inv = _inv_col(degp_ref[...])
        hs_ref[...] = x_ref[...] * inv

    return pl.pallas_call(
        kern,
        grid=(nb,),
        in_specs=[
            pl.BlockSpec((_NC * _NS, r), lambda i: (0, i)),
            pl.BlockSpec((r, d), lambda i: (i, 0)),
        ],
        out_specs=pl.BlockSpec((r, d), lambda i: (i, 0)),
        out_shape=jax.ShapeDtypeStruct((n, d), jnp.float32),
        compiler_params=_ARB,
    )


def _tc_layer1(n, d, nb, r):
    def kern(aggp_ref, x_ref, degp_ref, w_ref, b_ref, z_ref, hz_ref):
        p = aggp_ref[...]
        inv = _inv_col(degp_ref[...])
        x = x_ref[...]
        agg = (p[0] + p[1]) * inv + x * inv * inv
        z = jnp.maximum(
            jnp.dot(agg, w_ref[0], preferred_element_type=jnp.float32)
            + b_ref[0], 0.0)
        z_ref[...] = z[None]
        hz_ref[...] = (z * inv)[None]

    return pl.pallas_call(
        kern,
        grid=(2, nb),
        in_specs=[
            pl.BlockSpec((2, r, d), lambda br, i: (0, i, 0)),
            pl.BlockSpec((r, d), lambda br, i: (i, 0)),
            pl.BlockSpec((_NC * _NS, r), lambda br, i: (0, i)),
            pl.BlockSpec((1, d, d), lambda br, i: (br, 0, 0)),
            pl.BlockSpec((1, 1, d), lambda br, i: (br, 0, 0)),
        ],
        out_specs=[
            pl.BlockSpec((1, r, d), lambda br, i: (br, i, 0)),
            pl.BlockSpec((1, r, d), lambda br, i: (br, i, 0)),
        ],
        out_shape=[
            jax.ShapeDtypeStruct((2, n, d), jnp.float32),
            jax.ShapeDtypeStruct((2, n, d), jnp.float32),
        ],
        compiler_params=_ARB2,
    )


def _tc_final(n, d, nb, r):
    def kern(agg2_ref, z1_ref, degp_ref, batch_ref, w2_ref, b2_ref,
             m1w1_ref, m1b1_ref, m1w2_ref, m1b2_ref,
             m2w1_ref, m2b1_ref, m2w2_ref, m2b2_ref,
             h1_ref, h2_ref, g1_ref, g2_ref, gacc0, gacc1, cnt):
        i = pl.program_id(0)
        br = pl.program_id(1)
        nblocks = pl.num_programs(0)

        @pl.when((i == 0) & (br == 0))
        def _():
            gacc0[...] = jnp.zeros_like(gacc0)
            gacc1[...] = jnp.zeros_like(gacc1)
            cnt[...] = jnp.zeros_like(cnt)

        inv = _inv_col(degp_ref[...])
        z1 = z1_ref[0]
        agg = agg2_ref[0] * inv + z1 * inv * inv
        z2 = jnp.maximum(
            jnp.dot(agg, w2_ref[0], preferred_element_type=jnp.float32)
            + b2_ref[0], 0.0)

        t = jnp.maximum(
            jnp.dot(z2, m1w1_ref[...], preferred_element_type=jnp.float32)
            + m1b1_ref[...], 0.0)
        h = (jnp.dot(t, m1w2_ref[...], preferred_element_type=jnp.float32)
             + m1b2_ref[...])

        @pl.when(br == 0)
        def _():
            h1_ref[...] = h

        @pl.when(br == 1)
        def _():
            h2_ref[...] = h

        b = batch_ref[...]
        gids = lax.broadcasted_iota(jnp.int32, (_NG, r), 0)
        rows = lax.broadcasted_iota(jnp.int32, (1, r), 1) + i * r
        onehot = jnp.where((gids == b) & (rows < n), 1.0, 0.0)
        gpart = jnp.dot(onehot, z2, preferred_element_type=jnp.float32)

        @pl.when(br == 0)
        def _():
            gacc0[...] += gpart
            cnt[...] += jnp.broadcast_to(
                jnp.sum(onehot, axis=1, keepdims=True), (_NG, d))

        @pl.when(br == 1)
        def _():
            gacc1[...] += gpart

        @pl.when((i == nblocks - 1) & (br == 1))
        def _():
            c = jnp.maximum(cnt[...], 1.0)
            for g_ref, acc in ((g1_ref, gacc0), (g2_ref, gacc1)):
                gg = acc[...] / c
                gg = jnp.maximum(
                    jnp.dot(gg, m2w1_ref[...],
                            preferred_element_type=jnp.float32)
                    + m2b1_ref[...], 0.0)
                g_ref[...] = (jnp.dot(gg, m2w2_ref[...],
                                      preferred_element_type=jnp.float32)
                              + m2b2_ref[...])

    return pl.pallas_call(
        kern,
        grid=(nb, 2),
        in_specs=[
            pl.BlockSpec((1, r, d), lambda i, br: (br, i, 0)),
            pl.BlockSpec((1, r, d), lambda i, br: (br, i, 0)),
            pl.BlockSpec((_NC * _NS, r), lambda i, br: (0, i)),
            pl.BlockSpec((1, r), lambda i, br: (0, i)),
            pl.BlockSpec((1, d, d), lambda i, br: (br, 0, 0)),
            pl.BlockSpec((1, 1, d), lambda i, br: (br, 0, 0)),
            pl.BlockSpec((d, d), lambda i, br: (0, 0)),
            pl.BlockSpec((1, d), lambda i, br: (0, 0)),
            pl.BlockSpec((d, d), lambda i, br: (0, 0)),
            pl.BlockSpec((1, d), lambda i, br: (0, 0)),
            pl.BlockSpec((d, d), lambda i, br: (0, 0)),
            pl.BlockSpec((1, d), lambda i, br: (0, 0)),
            pl.BlockSpec((d, d), lambda i, br: (0, 0)),
            pl.BlockSpec((1, d), lambda i, br: (0, 0)),
        ],
        out_specs=[
            pl.BlockSpec((r, d), lambda i, br: (i, 0)),
            pl.BlockSpec((r, d), lambda i, br: (i, 0)),
            pl.BlockSpec((_NG, d), lambda i, br: (0, 0)),
            pl.BlockSpec((_NG, d), lambda i, br: (0, 0)),
        ],
        out_shape=[
            jax.ShapeDtypeStruct((n, d), jnp.float32),
            jax.ShapeDtypeStruct((n, d), jnp.float32),
            jax.ShapeDtypeStruct((_NG, d), jnp.float32),
            jax.ShapeDtypeStruct((_NG, d), jnp.float32),
        ],
        scratch_shapes=[
            pltpu.VMEM((_NG, d), jnp.float32),
            pltpu.VMEM((_NG, d), jnp.float32),
            pltpu.VMEM((_NG, d), jnp.float32),
        ],
        compiler_params=_ARB2,
    )


def kernel(x, edge_index, batch,
           gcn1_W1, gcn1_b1, gcn1_W2, gcn1_b2,
           gcn2_W1, gcn2_b1, gcn2_W2, gcn2_b2,
           mlp1_W1, mlp1_b1, mlp1_W2, mlp1_b2,
           mlp2_W1, mlp2_b1, mlp2_W2, mlp2_b2):
    n, d = x.shape
    e = edge_index.shape[1]
    r = 1024
    nb = pl.cdiv(n, r)
    rf = 2048
    nbf = pl.cdiv(n, rf)

    src = edge_index[0].astype(jnp.int32)
    dst = edge_index[1].astype(jnp.int32)
    dst3 = dst.reshape(-1, 2000 // _CH, _CH)

    degp = _sc_degree(n, e)(dst)
    hs = _tc_prep(n, d, nb, r)(degp, x)

    zero = jnp.zeros((n, d), jnp.float32)
    aggp = _sc_prop(n, d, e, False)(src, dst3, hs, zero)

    w1s = jnp.stack([gcn1_W1, gcn2_W1])
    b1s = jnp.stack([gcn1_b1, gcn2_b1])[:, None, :]
    z, hz = _tc_layer1(n, d, nb, r)(aggp, x, degp, w1s, b1s)

    agg2 = _sc_prop(n, d, e, True)(src, dst3, hz.reshape(2 * n, d), zero)

    w2s = jnp.stack([gcn1_W2, gcn2_W2])
    b2s = jnp.stack([gcn1_b2, gcn2_b2])[:, None, :]
    h1, h2, g1, g2 = _tc_final(n, d, nbf, rf)(
        agg2, z, degp, batch.astype(jnp.int32)[None], w2s, b2s,
        mlp1_W1, mlp1_b1[None], mlp1_W2, mlp1_b2[None],
        mlp2_W1, mlp2_b1[None], mlp2_W2, mlp2_b2[None])

    return h1, h2, g1, g2


# speedup vs baseline: 1.1010x; 1.0471x over previous
import functools

import jax
import jax.numpy as jnp
from jax import lax
from jax.experimental import pallas as pl
from jax.experimental.pallas import tpu as pltpu
from jax.experimental.pallas import tpu_sc as plsc

_NC = 2
_NS = 16
_CH = 80
_NG = 128


def _sc_degree(n, e):
    ept = e // (_NC * _NS)
    mesh = plsc.VectorSubcoreMesh(core_axis_name="c", subcore_axis_name="s")

    @functools.partial(
        pl.kernel,
        out_type=jax.ShapeDtypeStruct((_NC * _NS, n), jnp.float32),
        mesh=mesh,
        scratch_types=[
            pltpu.VMEM((ept,), jnp.int32),
            pltpu.VMEM((n,), jnp.float32),
        ],
        compiler_params=pltpu.CompilerParams(needs_layout_passes=False),
    )
    def deg_kernel(dst_hbm, out_hbm, dst_v, deg_v):
        c = lax.axis_index("c")
        s = lax.axis_index("s")
        wid = c * _NS + s
        pltpu.sync_copy(dst_hbm.at[pl.ds(wid * ept, ept)], dst_v)

        def zero_body(i, carry):
            for j in range(4):
                deg_v[pl.ds(i * 64 + j * 16, 16)] = jnp.zeros(
                    (16,), jnp.float32)
            return carry

        lax.fori_loop(0, n // 64, zero_body, 0)
        for j in range(n // 16 - (n // 64) * 4):
            deg_v[pl.ds((n // 64) * 64 + j * 16, 16)] = jnp.zeros(
                (16,), jnp.float32)

        ones = jnp.ones((16,), jnp.float32)

        def body(i, carry):
            for j in range(4):
                idx = dst_v[pl.ds(i * 64 + j * 16, 16)]
                plsc.addupdate_scatter(deg_v, [idx], ones)
            return carry

        lax.fori_loop(0, ept // 64, body, 0)
        for j in range(ept // 16 - (ept // 64) * 4):
            idx = dst_v[pl.ds((ept // 64) * 64 + j * 16, 16)]
            plsc.addupdate_scatter(deg_v, [idx], ones)
        pltpu.sync_copy(deg_v, out_hbm.at[wid])

    return deg_kernel


def _sc_prop(n, d, e, branch_split):
    ept = e // _NS if branch_split else e // (_NC * _NS)
    sb = 2000
    nsb = ept // sb
    ncs = sb // _CH
    npt = (n // (8 * _NS)) * 8
    rem = n - npt * _NS
    mesh = plsc.VectorSubcoreMesh(core_axis_name="c", subcore_axis_name="s")

    @functools.partial(
        pl.kernel,
        out_type=jax.ShapeDtypeStruct((_NC, n, d), jnp.float32),
        mesh=mesh,
        scratch_types=[
            pltpu.VMEM((sb,), jnp.int32),
            pltpu.VMEM((sb // _CH, _CH), jnp.int32),
            pltpu.VMEM((_CH, d), jnp.float32),
            pltpu.VMEM((_CH, d), jnp.float32),
            pltpu.VMEM((_CH, d), jnp.float32),
            pltpu.VMEM_SHARED((n, d), jnp.float32),
            pltpu.SemaphoreType.DMA,
            pltpu.SemaphoreType.DMA,
            pltpu.SemaphoreType.DMA,
            pltpu.SemaphoreType.DMA,
        ],
    )
    def prop_kernel(src_hbm, dst2_hbm, h_hbm, zero_hbm, out_hbm,
                    src_v, dst_v, buf0, buf1, buf2, acc,
                    sem0, sem1, sem2, semz):
        c = lax.axis_index("c")
        s = lax.axis_index("s")

        pltpu.async_copy(zero_hbm.at[pl.ds(s * npt, npt)],
                         acc.at[pl.ds(s * npt, npt)], semz)
        if rem:
            @pl.when(s == 0)
            def _():
                pltpu.async_copy(zero_hbm.at[pl.ds(npt * _NS, rem)],
                                 acc.at[pl.ds(npt * _NS, rem)], semz)

        if branch_split:
            base = s * ept
            sb_base = s * nsb
        else:
            base = (c * _NS + s) * ept
            sb_base = (c * _NS + s) * nsb
        off = c * n

        def zero_wait():
            pltpu.make_async_copy(zero_hbm.at[pl.ds(s * npt, npt)],
                                  acc.at[pl.ds(s * npt, npt)], semz).wait()
            if rem:
                @pl.when(s == 0)
                def _():
                    pltpu.make_async_copy(
                        zero_hbm.at[pl.ds(npt * _NS, rem)],
                        acc.at[pl.ds(npt * _NS, rem)], semz).wait()

        bufs = (buf0, buf1, buf2)
        sems = (sem0, sem1, sem2)
        nslot = len(bufs)

        def issue(k, p):
            pltpu.async_copy(h_hbm.at[src_v.at[pl.ds(k * _CH, _CH)]],
                             bufs[p], sems[p])

        def wait(p):
            pltpu.make_async_copy(h_hbm.at[pl.ds(0, _CH)], bufs[p],
                                  sems[p]).wait()

        def drain(k, p):
            wait(p)
            pltpu.sync_copy(bufs[p], acc.at[dst_v.at[k]], add=True)

        def sb_body(b, carry):
            pltpu.sync_copy(src_hbm.at[pl.ds(base + b * sb, sb)], src_v)
            pltpu.sync_copy(dst2_hbm.at[sb_base + b], dst_v)
            if branch_split:
                def offs_body(i, c2):
                    src_v[pl.ds(i * 16, 16)] = src_v[pl.ds(i * 16, 16)] + off
                    return c2

                lax.fori_loop(0, sb // 16, offs_body, 0)

            for p in range(nslot):
                issue(p, p)

            @pl.when(b == 0)
            def _():
                zero_wait()
                plsc.subcore_barrier()

            def body(k, c2):
                kk = nslot * k
                for p in range(nslot):
                    drain(kk + p, p)

                    @pl.when(kk + p + nslot < ncs)
                    def _():
                        issue(kk + p + nslot, p)

                return c2

            lax.fori_loop(0, ncs // nslot, body, 0)
            for t in range(ncs - (ncs // nslot) * nslot):
                drain((ncs // nslot) * nslot + t, t)
            return carry

        lax.fori_loop(0, nsb, sb_body, 0)

        plsc.subcore_barrier()
        pltpu.sync_copy(acc.at[pl.ds(s * npt, npt)],
                        out_hbm.at[c, pl.ds(s * npt, npt)])
        if rem:
            @pl.when(s == 0)
            def _():
                pltpu.sync_copy(acc.at[pl.ds(npt * _NS, rem)],
                                out_hbm.at[c, pl.ds(npt * _NS, rem)])

    return prop_kernel


_ARB = pltpu.CompilerParams(dimension_semantics=("arbitrary",))
_ARB2 = pltpu.CompilerParams(dimension_semantics=("arbitrary", "arbitrary"))


def _inv_col(degp):
    ones = jnp.ones((_NC * _NS, 1), jnp.float32)
    deg = lax.dot_general(degp, ones, (((0,), (0,)), ((), ())),
                          preferred_element_type=jnp.float32)
    return lax.rsqrt(deg + 1.0)


def _tc_prep(n, d, nb, r):
    def kern(degp_ref, x_ref, hs_ref):
        inv = _inv_col(degp_ref[...])
        hs_ref[...] = x_ref[...] * inv

    return pl.pallas_call(
        kern,
        grid=(nb,),
        in_specs=[
            pl.BlockSpec((_NC * _NS, r), lambda i: (0, i)),
            pl.BlockSpec((r, d), lambda i: (i, 0)),
        ],
        out_specs=pl.BlockSpec((r, d), lambda i: (i, 0)),
        out_shape=jax.ShapeDtypeStruct((n, d), jnp.float32),
        compiler_params=_ARB,
    )


def _tc_layer1(n, d, nb, r):
    def kern(aggp_ref, x_ref, degp_ref, w_ref, b_ref, z_ref, hz_ref):
        p = aggp_ref[...]
        inv = _inv_col(degp_ref[...])
        x = x_ref[...]
        agg = (p[0] + p[1]) * inv + x * inv * inv
        for br in range(2):
            z = jnp.maximum(
                jnp.dot(agg, w_ref[br], preferred_element_type=jnp.float32)
                + b_ref[br], 0.0)
            z_ref[br] = z
            hz_ref[br] = z * inv

    return pl.pallas_call(
        kern,
        grid=(nb,),
        in_specs=[
            pl.BlockSpec((2, r, d), lambda i: (0, i, 0)),
            pl.BlockSpec((r, d), lambda i: (i, 0)),
            pl.BlockSpec((_NC * _NS, r), lambda i: (0, i)),
            pl.BlockSpec((2, d, d), lambda i: (0, 0, 0)),
            pl.BlockSpec((2, 1, d), lambda i: (0, 0, 0)),
        ],
        out_specs=[
            pl.BlockSpec((2, r, d), lambda i: (0, i, 0)),
            pl.BlockSpec((2, r, d), lambda i: (0, i, 0)),
        ],
        out_shape=[
            jax.ShapeDtypeStruct((2, n, d), jnp.float32),
            jax.ShapeDtypeStruct((2, n, d), jnp.float32),
        ],
        compiler_params=_ARB,
    )


def _tc_final(n, d, nb, r):
    def kern(agg2_ref, z1_ref, degp_ref, batch_ref, w2_ref, b2_ref,
             m1w1_ref, m1b1_ref, m1w2_ref, m1b2_ref,
             m2w1_ref, m2b1_ref, m2w2_ref, m2b2_ref,
             h1_ref, h2_ref, g1_ref, g2_ref, gacc0, gacc1, cnt):
        i = pl.program_id(0)
        nblocks = pl.num_programs(0)

        @pl.when(i == 0)
        def _():
            gacc0[...] = jnp.zeros_like(gacc0)
            gacc1[...] = jnp.zeros_like(gacc1)
            cnt[...] = jnp.zeros_like(cnt)

        inv = _inv_col(degp_ref[...])

        b = batch_ref[...]
        gids = lax.broadcasted_iota(jnp.int32, (_NG, r), 0)
        rows = lax.broadcasted_iota(jnp.int32, (1, r), 1) + i * r
        onehot = jnp.where((gids == b) & (rows < n), 1.0, 0.0)
        cnt[...] += jnp.broadcast_to(
            jnp.sum(onehot, axis=1, keepdims=True), (_NG, d))

        for br, (h_ref, gacc) in enumerate(((h1_ref, gacc0),
                                            (h2_ref, gacc1))):
            agg = agg2_ref[br] * inv + z1_ref[br] * inv * inv
            z2 = jnp.maximum(
                jnp.dot(agg, w2_ref[br], preferred_element_type=jnp.float32)
                + b2_ref[br], 0.0)
            t = jnp.maximum(
                jnp.dot(z2, m1w1_ref[...],
                        preferred_element_type=jnp.float32)
                + m1b1_ref[...], 0.0)
            h_ref[...] = (jnp.dot(t, m1w2_ref[...],
                                  preferred_element_type=jnp.float32)
                          + m1b2_ref[...])
            gacc[...] += jnp.dot(onehot, z2,
                                 preferred_element_type=jnp.float32)

        @pl.when(i == nblocks - 1)
        def _():
            c = jnp.maximum(cnt[...], 1.0)
            for g_ref, acc in ((g1_ref, gacc0), (g2_ref, gacc1)):
                gg = acc[...] / c
                gg = jnp.maximum(
                    jnp.dot(gg, m2w1_ref[...],
                            preferred_element_type=jnp.float32)
                    + m2b1_ref[...], 0.0)
                g_ref[...] = (jnp.dot(gg, m2w2_ref[...],
                                      preferred_element_type=jnp.float32)
                              + m2b2_ref[...])

    return pl.pallas_call(
        kern,
        grid=(nb,),
        in_specs=[
            pl.BlockSpec((2, r, d), lambda i: (0, i, 0)),
            pl.BlockSpec((2, r, d), lambda i: (0, i, 0)),
            pl.BlockSpec((_NC * _NS, r), lambda i: (0, i)),
            pl.BlockSpec((1, r), lambda i: (0, i)),
            pl.BlockSpec((2, d, d), lambda i: (0, 0, 0)),
            pl.BlockSpec((2, 1, d), lambda i: (0, 0, 0)),
            pl.BlockSpec((d, d), lambda i: (0, 0)),
            pl.BlockSpec((1, d), lambda i: (0, 0)),
            pl.BlockSpec((d, d), lambda i: (0, 0)),
            pl.BlockSpec((1, d), lambda i: (0, 0)),
            pl.BlockSpec((d, d), lambda i: (0, 0)),
            pl.BlockSpec((1, d), lambda i: (0, 0)),
            pl.BlockSpec((d, d), lambda i: (0, 0)),
            pl.BlockSpec((1, d), lambda i: (0, 0)),
        ],
        out_specs=[
            pl.BlockSpec((r, d), lambda i: (i, 0)),
            pl.BlockSpec((r, d), lambda i: (i, 0)),
            pl.BlockSpec((_NG, d), lambda i: (0, 0)),
            pl.BlockSpec((_NG, d), lambda i: (0, 0)),
        ],
        out_shape=[
            jax.ShapeDtypeStruct((n, d), jnp.float32),
            jax.ShapeDtypeStruct((n, d), jnp.float32),
            jax.ShapeDtypeStruct((_NG, d), jnp.float32),
            jax.ShapeDtypeStruct((_NG, d), jnp.float32),
        ],
        scratch_shapes=[
            pltpu.VMEM((_NG, d), jnp.float32),
            pltpu.VMEM((_NG, d), jnp.float32),
            pltpu.VMEM((_NG, d), jnp.float32),
        ],
        compiler_params=_ARB,
    )


def kernel(x, edge_index, batch,
           gcn1_W1, gcn1_b1, gcn1_W2, gcn1_b2,
           gcn2_W1, gcn2_b1, gcn2_W2, gcn2_b2,
           mlp1_W1, mlp1_b1, mlp1_W2, mlp1_b2,
           mlp2_W1, mlp2_b1, mlp2_W2, mlp2_b2):
    n, d = x.shape
    e = edge_index.shape[1]
    r = 1024
    nb = pl.cdiv(n, r)
    rf = 2048
    nbf = pl.cdiv(n, rf)

    src = edge_index[0].astype(jnp.int32)
    dst = edge_index[1].astype(jnp.int32)
    dst3 = dst.reshape(-1, 2000 // _CH, _CH)

    degp = _sc_degree(n, e)(dst)
    hs = _tc_prep(n, d, nb, r)(degp, x)

    zero = jnp.zeros((n, d), jnp.float32)
    aggp = _sc_prop(n, d, e, False)(src, dst3, hs, zero)

    w1s = jnp.stack([gcn1_W1, gcn2_W1])
    b1s = jnp.stack([gcn1_b1, gcn2_b1])[:, None, :]
    z, hz = _tc_layer1(n, d, nb, r)(aggp, x, degp, w1s, b1s)

    agg2 = _sc_prop(n, d, e, True)(src, dst3, hz.reshape(2 * n, d), zero)

    w2s = jnp.stack([gcn1_W2, gcn2_W2])
    b2s = jnp.stack([gcn1_b2, gcn2_b2])[:, None, :]
    h1, h2, g1, g2 = _tc_final(n, d, nbf, rf)(
        agg2, z, degp, batch.astype(jnp.int32)[None], w2s, b2s,
        mlp1_W1, mlp1_b1[None], mlp1_W2, mlp1_b2[None],
        mlp2_W1, mlp2_b1[None], mlp2_W2, mlp2_b2[None])

    return h1, h2, g1, g2
